# baseline (speedup 1.0000x reference)
import functools

import jax
import jax.numpy as jnp
from jax import lax
from jax.experimental import pallas as pl
from jax.experimental.pallas import tpu as pltpu
from jax.experimental.pallas import tpu_sc as plsc

VOCAB = 1000000
EMBED_DIM = 64
PAD_LEN = 200
BATCH = 4096
CLASS_NUM = 16

_D = EMBED_DIM
_L = PAD_LEN
_CH0 = 128
_CH1 = _L - _CH0

_NC = 2
_NS = 16
_NW = _NC * _NS
_ROWS_PER_W = BATCH // _NW
_UNROLL = 8


def _fire_gather(table_hbm, tex_v, buf, sem, r):
    pltpu.make_async_copy(
        table_hbm.at[tex_v.at[r, pl.ds(0, _CH0)]],
        buf.at[pl.ds(0, _CH0)], sem).start()
    pltpu.make_async_copy(
        table_hbm.at[tex_v.at[r, pl.ds(_CH0, _CH1)]],
        buf.at[pl.ds(_CH0, _CH1)], sem).start()


def _drain_gather(table_hbm, buf, sem):
    pltpu.make_async_copy(table_hbm.at[pl.ds(0, _L)], buf, sem).wait()


def _reduce_rows(buf):
    def body(j, accs):
        a0, a1, a2, a3 = accs
        for k in range(_UNROLL):
            r = j * _UNROLL + k
            a0 = a0 + buf[r, pl.ds(0, 16)]
            a1 = a1 + buf[r, pl.ds(16, 16)]
            a2 = a2 + buf[r, pl.ds(32, 16)]
            a3 = a3 + buf[r, pl.ds(48, 16)]
        return (a0, a1, a2, a3)

    z = jnp.zeros((16,), jnp.float32)
    return lax.fori_loop(0, _L // _UNROLL, body, (z, z, z, z))


@functools.partial(
    pl.kernel,
    mesh=plsc.VectorSubcoreMesh(core_axis_name="c", subcore_axis_name="s"),
    out_type=jax.ShapeDtypeStruct((BATCH, _D), jnp.float32),
    compiler_params=pltpu.CompilerParams(use_tc_tiling_on_sc=False),
    scratch_types=[
        pltpu.VMEM((_ROWS_PER_W, _L), jnp.int32),
        pltpu.VMEM((_L, _D), jnp.float32),
        pltpu.VMEM((_L, _D), jnp.float32),
        pltpu.VMEM((_ROWS_PER_W, _D), jnp.float32),
        pltpu.SemaphoreType.DMA,
        pltpu.SemaphoreType.DMA,
    ],
)
def _sc_lookup_sum(texts_hbm, table_hbm, out_hbm, tex_v, buf0, buf1,
                   sum_v, sem0, sem1):
    wid = lax.axis_index("s") * _NC + lax.axis_index("c")
    base = wid * _ROWS_PER_W
    pltpu.sync_copy(texts_hbm.at[pl.ds(base, _ROWS_PER_W)], tex_v)

    _fire_gather(table_hbm, tex_v, buf0, sem0, 0)
    _fire_gather(table_hbm, tex_v, buf1, sem1, 1)

    def body(i, carry):
        del carry
        r0 = 2 * i
        _drain_gather(table_hbm, buf0, sem0)
        a0, a1, a2, a3 = _reduce_rows(buf0)
        sum_v[r0, pl.ds(0, 16)] = a0
        sum_v[r0, pl.ds(16, 16)] = a1
        sum_v[r0, pl.ds(32, 16)] = a2
        sum_v[r0, pl.ds(48, 16)] = a3

        @pl.when(i < _ROWS_PER_W // 2 - 1)
        def _():
            _fire_gather(table_hbm, tex_v, buf0, sem0, r0 + 2)

        _drain_gather(table_hbm, buf1, sem1)
        b0, b1, b2, b3 = _reduce_rows(buf1)
        sum_v[r0 + 1, pl.ds(0, 16)] = b0
        sum_v[r0 + 1, pl.ds(16, 16)] = b1
        sum_v[r0 + 1, pl.ds(32, 16)] = b2
        sum_v[r0 + 1, pl.ds(48, 16)] = b3

        @pl.when(i < _ROWS_PER_W // 2 - 1)
        def _():
            _fire_gather(table_hbm, tex_v, buf1, sem1, r0 + 3)

        return 0

    lax.fori_loop(0, _ROWS_PER_W // 2, body, 0)
    pltpu.sync_copy(sum_v, out_hbm.at[pl.ds(base, _ROWS_PER_W)])


def _fc_body(x_ref, w_ref, b_ref, o_ref):
    o_ref[...] = (
        jnp.dot(x_ref[...], w_ref[...], preferred_element_type=jnp.float32)
        + b_ref[...]
    )


_fc_call = pl.pallas_call(
    _fc_body,
    out_shape=jax.ShapeDtypeStruct((BATCH, 128), jnp.float32),
)


def kernel(texts, table, fc_w, fc_b):
    sums = _sc_lookup_sum(texts.astype(jnp.int32), table)
    w_t = jnp.transpose(fc_w) * jnp.float32(1.0 / _L)
    w_pad = jnp.pad(w_t, ((0, 0), (0, 128 - CLASS_NUM)))
    b_pad = jnp.pad(fc_b, (0, 128 - CLASS_NUM)).reshape(1, 128)
    out = _fc_call(sums, w_pad, b_pad)
    return out[:, :CLASS_NUM]
